# Initial kernel scaffold; baseline (speedup 1.0000x reference)
import jax
import jax.numpy as jnp
from jax.experimental import pallas as pl


def kernel(x, edge_index, edge_weight, W, conv_bias, lstm_w_ih, lstm_w_hh, lstm_b_ih, lstm_b_hh, lin_w, lin_b):
    raise NotImplementedError("write your pallas kernel here")


# speedup vs baseline: 9.2366x; 9.2366x over previous
import functools

import jax
import jax.numpy as jnp
from jax import lax
from jax.experimental import pallas as pl
from jax.experimental.pallas import tpu as pltpu
from jax.experimental.pallas import tpu_sc as plsc

N = 10000
NP = 10240
D = 256
H = 128
E = 160000
NC, NS, L = 2, 16, 16
NW = NC * NS
CHA = 112
NCHA = 45
EPT_A = CHA * NCHA
EA = NW * EPT_A
EPT_B = E // NS
CH = 80
NCHUNK = EPT_B // CH
CPS = 25
NSB = NCHUNK // CPS
ROWS_PT = NP // NS

_mesh = plsc.VectorSubcoreMesh(
    core_axis_name="c", subcore_axis_name="s", num_cores=NC, num_subcores=NS)


@functools.partial(
    pl.kernel,
    out_type=jax.ShapeDtypeStruct((NC, NP, L), jnp.float32),
    mesh=_mesh,
    scratch_types=[
        pltpu.VMEM((NCHA, CHA), jnp.int32),
        pltpu.VMEM((NCHA * CHA,), jnp.float32),
        pltpu.VMEM((CHA, L), jnp.float32),
        pltpu.VMEM_SHARED((NP, L), jnp.float32),
    ],
    compiler_params=pltpu.CompilerParams(needs_layout_passes=False),
)
def _deg_kernel(dst_hbm, w_hbm, out_hbm, dst_v, w_v, wide, deg_sh):
    cid = lax.axis_index("c")
    sid = lax.axis_index("s")
    wid = sid * NC + cid
    pltpu.sync_copy(dst_hbm.at[wid], dst_v)
    pltpu.sync_copy(w_hbm.at[wid], w_v)
    zero = jnp.zeros((L,), jnp.float32)

    def zrow(r, carry):
        wide[r] = zero
        return carry

    lax.fori_loop(0, CHA, zrow, None)
    for k in range(ROWS_PT // CHA + 1):
        rows = min(CHA, ROWS_PT - k * CHA)
        if rows > 0:
            pltpu.sync_copy(wide.at[pl.ds(0, rows)],
                            deg_sh.at[pl.ds(sid * ROWS_PT + k * CHA, rows)])
    plsc.subcore_barrier()

    def chunk(c, carry):
        def fill(e, inner):
            wv = plsc.load_gather(w_v, [jnp.full((L,), c * CHA + e, jnp.int32)])
            wide[e] = wv
            return inner

        lax.fori_loop(0, CHA, fill, None)
        pltpu.sync_copy(wide, deg_sh.at[dst_v.at[c]], add=True)
        return carry

    lax.fori_loop(0, NCHA, chunk, None)
    plsc.subcore_barrier()
    pltpu.sync_copy(deg_sh.at[pl.ds(sid * ROWS_PT, ROWS_PT)],
                    out_hbm.at[cid, pl.ds(sid * ROWS_PT, ROWS_PT)])


def _prologue_body(w_ref, wih_ref, b_ref, degp_ref, x_ref, yt_ref, dis_ref):
    gates = lax.dot_general(
        w_ref[...], wih_ref[...], (((1,), (1,)), ((), ())),
        preferred_element_type=jnp.float32) + b_ref[...]
    i_ = jax.nn.sigmoid(gates[:, 0 * D:1 * D])
    g_ = jnp.tanh(gates[:, 2 * D:3 * D])
    o_ = jax.nn.sigmoid(gates[:, 3 * D:4 * D])
    w_ev = o_ * jnp.tanh(i_ * g_)
    deg = degp_ref[0, :, 0:1] + degp_ref[1, :, 0:1] + 2.0
    dis = lax.rsqrt(deg)
    dis_ref[...] = dis
    xs = x_ref[...] * dis
    yt_ref[0] = jnp.dot(xs, w_ev[:, :H], preferred_element_type=jnp.float32)
    yt_ref[1] = jnp.dot(xs, w_ev[:, H:], preferred_element_type=jnp.float32)


@functools.partial(
    pl.kernel,
    out_type=jax.ShapeDtypeStruct((NC, NP, H), jnp.float32),
    mesh=_mesh,
    scratch_types=[
        pltpu.VMEM((CPS, CH), jnp.int32),
        pltpu.VMEM((CPS, CH), jnp.int32),
        pltpu.VMEM((CPS * CH,), jnp.float32),
        pltpu.VMEM((CH, H), jnp.float32),
        pltpu.VMEM_SHARED((NP, H), jnp.float32),
        pltpu.SemaphoreType.DMA,
    ],
    compiler_params=pltpu.CompilerParams(needs_layout_passes=False),
)
def _spmm_kernel(y0, y1, srcs, dsts, ws, out_hbm,
                 src_v, dst_v, w_v, buf, agg_sh, sem):
    cid = lax.axis_index("c")
    sid = lax.axis_index("s")
    zero = jnp.zeros((L,), jnp.float32)

    def zrow(r, carry):
        for j in range(H // L):
            buf[r, pl.ds(j * L, L)] = zero
        return carry

    lax.fori_loop(0, CH, zrow, None)
    for k in range(ROWS_PT // CH):
        pltpu.sync_copy(buf, agg_sh.at[pl.ds(sid * ROWS_PT + k * CH, CH)])
    plsc.subcore_barrier()

    def sblock(sb, carry):
        pltpu.sync_copy(srcs.at[sid, sb], src_v)
        pltpu.sync_copy(dsts.at[sid, sb], dst_v)
        pltpu.sync_copy(ws.at[sid, sb], w_v)

        def chunk(c, carry2):
            @pl.when(cid == 0)
            def _g0():
                pltpu.async_copy(y0.at[src_v.at[c]], buf, sem).wait()

            @pl.when(cid == 1)
            def _g1():
                pltpu.async_copy(y1.at[src_v.at[c]], buf, sem).wait()

            def scale(e, inner):
                wv = plsc.load_gather(
                    w_v, [jnp.full((L,), c * CH + e, jnp.int32)])
                for j in range(H // L):
                    buf[e, pl.ds(j * L, L)] = buf[e, pl.ds(j * L, L)] * wv
                return inner

            lax.fori_loop(0, CH, scale, None)
            pltpu.sync_copy(buf, agg_sh.at[dst_v.at[c]], add=True)
            return carry2

        lax.fori_loop(0, CPS, chunk, None)
        return carry

    lax.fori_loop(0, NSB, sblock, None)
    plsc.subcore_barrier()
    pltpu.sync_copy(agg_sh.at[pl.ds(sid * ROWS_PT, ROWS_PT)],
                    out_hbm.at[cid, pl.ds(sid * ROWS_PT, ROWS_PT)])


def _epilogue_body(agg_ref, yt_ref, dis_ref, bias_ref, lw_ref, lb_ref, out_ref):
    disf = dis_ref[...]
    h0 = jnp.maximum(
        disf * agg_ref[0] + (2.0 * disf) * yt_ref[0] + bias_ref[0:1, :], 0.0)
    h1 = jnp.maximum(
        disf * agg_ref[1] + (2.0 * disf) * yt_ref[1] + bias_ref[1:2, :], 0.0)
    h = jnp.concatenate([h0, h1], axis=1)
    out = jnp.dot(h, lw_ref[...], preferred_element_type=jnp.float32)
    out_ref[...] = out[:, 0:1] + lb_ref[0, 0]


def kernel(x, edge_index, edge_weight, W, conv_bias,
           lstm_w_ih, lstm_w_hh, lstm_b_ih, lstm_b_hh, lin_w, lin_b):
    src = edge_index[0].astype(jnp.int32)
    dst = edge_index[1].astype(jnp.int32)
    ew = edge_weight.astype(jnp.float32)
    x_p = jnp.pad(x, ((0, NP - N), (0, 0)))

    dst_a = jnp.full((EA,), N, jnp.int32).at[:E].set(dst)
    ew_a = jnp.zeros((EA,), jnp.float32).at[:E].set(ew)
    deg_parts = _deg_kernel(dst_a.reshape(NW, NCHA, CHA),
                            ew_a.reshape(NW, NCHA * CHA))

    b = (lstm_b_ih + lstm_b_hh).reshape(1, 4 * D)
    yt, dis = pl.pallas_call(
        _prologue_body,
        out_shape=[
            jax.ShapeDtypeStruct((2, NP, H), jnp.float32),
            jax.ShapeDtypeStruct((NP, 1), jnp.float32),
        ],
    )(W, lstm_w_ih, b, deg_parts, x_p)

    agg = _spmm_kernel(yt[0], yt[1],
                       src.reshape(NS, NSB, CPS, CH),
                       dst.reshape(NS, NSB, CPS, CH),
                       ew.reshape(NS, NSB, CPS * CH))

    out = pl.pallas_call(
        _epilogue_body,
        out_shape=jax.ShapeDtypeStruct((NP, 1), jnp.float32),
    )(agg, yt, dis, conv_bias.reshape(2, H),
      jnp.zeros((D, H), jnp.float32).at[:, 0].set(lin_w[0]),
      lin_b.reshape(1, 1))
    return out[:N]


# speedup vs baseline: 13.2184x; 1.4311x over previous
import functools

import jax
import jax.numpy as jnp
from jax import lax
from jax.experimental import pallas as pl
from jax.experimental.pallas import tpu as pltpu
from jax.experimental.pallas import tpu_sc as plsc

N = 10000
NP = 10240
D = 256
H = 128
E = 160000
NC, NS, L = 2, 16, 16
NW = NC * NS
CHA = 112
NCHA = 45
EPT_A = CHA * NCHA
EA = NW * EPT_A
EPT_B = E // NS
CH = 100
NCHUNK = EPT_B // CH
CPS = 20
NSB = NCHUNK // CPS
ROWS_PT = NP // NS

_mesh = plsc.VectorSubcoreMesh(
    core_axis_name="c", subcore_axis_name="s", num_cores=NC, num_subcores=NS)


@functools.partial(
    pl.kernel,
    out_type=jax.ShapeDtypeStruct((NC, NP, L), jnp.float32),
    mesh=_mesh,
    scratch_types=[
        pltpu.VMEM((NCHA, CHA), jnp.int32),
        pltpu.VMEM((NCHA * CHA,), jnp.float32),
        pltpu.VMEM((CHA, L), jnp.float32),
        pltpu.VMEM_SHARED((NP, L), jnp.float32),
    ],
    compiler_params=pltpu.CompilerParams(needs_layout_passes=False),
)
def _deg_kernel(dst_hbm, w_hbm, out_hbm, dst_v, w_v, wide, deg_sh):
    cid = lax.axis_index("c")
    sid = lax.axis_index("s")
    wid = sid * NC + cid
    pltpu.sync_copy(dst_hbm.at[wid], dst_v)
    pltpu.sync_copy(w_hbm.at[wid], w_v)
    zero = jnp.zeros((L,), jnp.float32)

    def zrow(r, carry):
        wide[r] = zero
        return carry

    lax.fori_loop(0, CHA, zrow, None)
    for k in range(ROWS_PT // CHA + 1):
        rows = min(CHA, ROWS_PT - k * CHA)
        if rows > 0:
            pltpu.sync_copy(wide.at[pl.ds(0, rows)],
                            deg_sh.at[pl.ds(sid * ROWS_PT + k * CHA, rows)])
    plsc.subcore_barrier()

    def chunk(c, carry):
        def fill(e, inner):
            wv = plsc.load_gather(w_v, [jnp.full((L,), c * CHA + e, jnp.int32)])
            wide[e] = wv
            return inner

        lax.fori_loop(0, CHA, fill, None)
        pltpu.sync_copy(wide, deg_sh.at[dst_v.at[c]], add=True)
        return carry

    lax.fori_loop(0, NCHA, chunk, None)
    plsc.subcore_barrier()
    pltpu.sync_copy(deg_sh.at[pl.ds(sid * ROWS_PT, ROWS_PT)],
                    out_hbm.at[cid, pl.ds(sid * ROWS_PT, ROWS_PT)])


def _prologue_body(w_ref, wih_ref, b_ref, degp_ref, x_ref, yt_ref, dis_ref):
    gates = lax.dot_general(
        w_ref[...], wih_ref[...], (((1,), (1,)), ((), ())),
        preferred_element_type=jnp.float32) + b_ref[...]
    i_ = jax.nn.sigmoid(gates[:, 0 * D:1 * D])
    g_ = jnp.tanh(gates[:, 2 * D:3 * D])
    o_ = jax.nn.sigmoid(gates[:, 3 * D:4 * D])
    w_ev = o_ * jnp.tanh(i_ * g_)
    deg = degp_ref[0, :, 0:1] + degp_ref[1, :, 0:1] + 2.0
    dis = lax.rsqrt(deg)
    dis_ref[...] = dis
    xs = x_ref[...] * dis
    yt_ref[0] = jnp.dot(xs, w_ev[:, :H], preferred_element_type=jnp.float32)
    yt_ref[1] = jnp.dot(xs, w_ev[:, H:], preferred_element_type=jnp.float32)


@functools.partial(
    pl.kernel,
    out_type=jax.ShapeDtypeStruct((NC, NP, H), jnp.float32),
    mesh=_mesh,
    scratch_types=[
        pltpu.VMEM((CPS, CH), jnp.int32),
        pltpu.VMEM((CPS, CH), jnp.int32),
        pltpu.VMEM((CPS * CH,), jnp.float32),
        pltpu.VMEM((CH, H), jnp.float32),
        pltpu.VMEM((CH, H), jnp.float32),
        pltpu.VMEM_SHARED((NP, H), jnp.float32),
        pltpu.SemaphoreType.DMA,
        pltpu.SemaphoreType.DMA,
    ],
    compiler_params=pltpu.CompilerParams(needs_layout_passes=False),
)
def _spmm_kernel(y0, y1, srcs, dsts, ws, out_hbm,
                 src_v, dst_v, w_v, buf_a, buf_b, agg_sh, sem_a, sem_b):
    cid = lax.axis_index("c")
    sid = lax.axis_index("s")
    zero = jnp.zeros((L,), jnp.float32)

    def zrow(r, carry):
        for j in range(H // L):
            buf_a[r, pl.ds(j * L, L)] = zero
        return carry

    lax.fori_loop(0, CH, zrow, None)
    off = 0
    while off < ROWS_PT:
        rows = min(CH, ROWS_PT - off)
        pltpu.sync_copy(buf_a.at[pl.ds(0, rows)],
                        agg_sh.at[pl.ds(sid * ROWS_PT + off, rows)])
        off += rows
    plsc.subcore_barrier()

    def gather(c, buf, sem):
        @pl.when(cid == 0)
        def _g0():
            pltpu.async_copy(y0.at[src_v.at[c]], buf, sem)

        @pl.when(cid == 1)
        def _g1():
            pltpu.async_copy(y1.at[src_v.at[c]], buf, sem)

    def sblock(sb, carry):
        pltpu.sync_copy(srcs.at[sid, sb], src_v)
        pltpu.sync_copy(dsts.at[sid, sb], dst_v)
        pltpu.sync_copy(ws.at[sid, sb], w_v)
        gather(0, buf_a, sem_a)

        def grp(g, carry2):
            for b in range(2):
                c = g * 2 + b
                buf, sem = (buf_a, sem_a) if b == 0 else (buf_b, sem_b)
                nbuf, nsem = (buf_b, sem_b) if b == 0 else (buf_a, sem_a)

                @pl.when(c + 1 < CPS)
                def _pref():
                    gather(c + 1, nbuf, nsem)

                pltpu.make_async_copy(y0.at[src_v.at[c]], buf, sem).wait()

                def scale(e, inner):
                    wv = plsc.load_gather(
                        w_v, [jnp.full((L,), c * CH + e, jnp.int32)])
                    for j in range(H // L):
                        buf[e, pl.ds(j * L, L)] = buf[e, pl.ds(j * L, L)] * wv
                    return inner

                lax.fori_loop(0, CH, scale, None)
                pltpu.sync_copy(buf, agg_sh.at[dst_v.at[c]], add=True)
            return carry2

        lax.fori_loop(0, CPS // 2, grp, None)
        return carry

    lax.fori_loop(0, NSB, sblock, None)
    plsc.subcore_barrier()
    pltpu.sync_copy(agg_sh.at[pl.ds(sid * ROWS_PT, ROWS_PT)],
                    out_hbm.at[cid, pl.ds(sid * ROWS_PT, ROWS_PT)])


def _epilogue_body(agg_ref, yt_ref, dis_ref, bias_ref, lw_ref, lb_ref, out_ref):
    disf = dis_ref[...]
    h0 = jnp.maximum(
        disf * agg_ref[0] + (2.0 * disf) * yt_ref[0] + bias_ref[0:1, :], 0.0)
    h1 = jnp.maximum(
        disf * agg_ref[1] + (2.0 * disf) * yt_ref[1] + bias_ref[1:2, :], 0.0)
    h = jnp.concatenate([h0, h1], axis=1)
    out = jnp.dot(h, lw_ref[...], preferred_element_type=jnp.float32)
    out_ref[...] = out[:, 0:1] + lb_ref[0, 0]


def kernel(x, edge_index, edge_weight, W, conv_bias,
           lstm_w_ih, lstm_w_hh, lstm_b_ih, lstm_b_hh, lin_w, lin_b):
    src = edge_index[0].astype(jnp.int32)
    dst = edge_index[1].astype(jnp.int32)
    ew = edge_weight.astype(jnp.float32)
    x_p = jnp.pad(x, ((0, NP - N), (0, 0)))

    dst_a = jnp.full((EA,), N, jnp.int32).at[:E].set(dst)
    ew_a = jnp.zeros((EA,), jnp.float32).at[:E].set(ew)
    deg_parts = _deg_kernel(dst_a.reshape(NW, NCHA, CHA),
                            ew_a.reshape(NW, NCHA * CHA))

    b = (lstm_b_ih + lstm_b_hh).reshape(1, 4 * D)
    yt, dis = pl.pallas_call(
        _prologue_body,
        out_shape=[
            jax.ShapeDtypeStruct((2, NP, H), jnp.float32),
            jax.ShapeDtypeStruct((NP, 1), jnp.float32),
        ],
    )(W, lstm_w_ih, b, deg_parts, x_p)

    agg = _spmm_kernel(yt[0], yt[1],
                       src.reshape(NS, NSB, CPS, CH),
                       dst.reshape(NS, NSB, CPS, CH),
                       ew.reshape(NS, NSB, CPS * CH))

    out = pl.pallas_call(
        _epilogue_body,
        out_shape=jax.ShapeDtypeStruct((NP, 1), jnp.float32),
    )(agg, yt, dis, conv_bias.reshape(2, H),
      jnp.zeros((D, H), jnp.float32).at[:, 0].set(lin_w[0]),
      lin_b.reshape(1, 1))
    return out[:N]


# speedup vs baseline: 13.7228x; 1.0382x over previous
import functools

import jax
import jax.numpy as jnp
from jax import lax
from jax.experimental import pallas as pl
from jax.experimental.pallas import tpu as pltpu
from jax.experimental.pallas import tpu_sc as plsc

N = 10000
NP = 10240
D = 256
H = 128
E = 160000
NC, NS, L = 2, 16, 16
NW = NC * NS
CHA = 112
NCHA = 45
EPT_A = CHA * NCHA
EA = NW * EPT_A
EPT_B = E // NS
CH = 100
NCHUNK = EPT_B // CH
CPS = 20
NSB = NCHUNK // CPS
ROWS_PT = NP // NS

_mesh = plsc.VectorSubcoreMesh(
    core_axis_name="c", subcore_axis_name="s", num_cores=NC, num_subcores=NS)


@functools.partial(
    pl.kernel,
    out_type=jax.ShapeDtypeStruct((NC, NP, L), jnp.float32),
    mesh=_mesh,
    scratch_types=[
        pltpu.VMEM((NCHA, CHA), jnp.int32),
        pltpu.VMEM((NCHA * CHA,), jnp.float32),
        pltpu.VMEM((CHA, L), jnp.float32),
        pltpu.VMEM_SHARED((NP, L), jnp.float32),
    ],
    compiler_params=pltpu.CompilerParams(needs_layout_passes=False),
)
def _deg_kernel(dst_hbm, w_hbm, out_hbm, dst_v, w_v, wide, deg_sh):
    cid = lax.axis_index("c")
    sid = lax.axis_index("s")
    wid = sid * NC + cid
    pltpu.sync_copy(dst_hbm.at[wid], dst_v)
    pltpu.sync_copy(w_hbm.at[wid], w_v)
    zero = jnp.zeros((L,), jnp.float32)

    def zrow(r, carry):
        wide[r] = zero
        return carry

    lax.fori_loop(0, CHA, zrow, None)
    for k in range(ROWS_PT // CHA + 1):
        rows = min(CHA, ROWS_PT - k * CHA)
        if rows > 0:
            pltpu.sync_copy(wide.at[pl.ds(0, rows)],
                            deg_sh.at[pl.ds(sid * ROWS_PT + k * CHA, rows)])
    plsc.subcore_barrier()

    def chunk(c, carry):
        def fill(e, inner):
            wv = plsc.load_gather(w_v, [jnp.full((L,), c * CHA + e, jnp.int32)])
            wide[e] = wv
            return inner

        lax.fori_loop(0, CHA, fill, None)
        pltpu.sync_copy(wide, deg_sh.at[dst_v.at[c]], add=True)
        return carry

    lax.fori_loop(0, NCHA, chunk, None)
    plsc.subcore_barrier()
    pltpu.sync_copy(deg_sh.at[pl.ds(sid * ROWS_PT, ROWS_PT)],
                    out_hbm.at[cid, pl.ds(sid * ROWS_PT, ROWS_PT)])


def _prologue_body(w_ref, wih_ref, b_ref, degp_ref, x_ref, yt_ref, dis_ref):
    gates = lax.dot_general(
        w_ref[...], wih_ref[...], (((1,), (1,)), ((), ())),
        preferred_element_type=jnp.float32) + b_ref[...]
    i_ = jax.nn.sigmoid(gates[:, 0 * D:1 * D])
    g_ = jnp.tanh(gates[:, 2 * D:3 * D])
    o_ = jax.nn.sigmoid(gates[:, 3 * D:4 * D])
    w_ev = o_ * jnp.tanh(i_ * g_)
    deg = degp_ref[0, :, 0:1] + degp_ref[1, :, 0:1] + 2.0
    dis = lax.rsqrt(deg)
    dis_ref[...] = dis
    xs = x_ref[...] * dis
    yt_ref[0] = jnp.dot(xs, w_ev[:, :H], preferred_element_type=jnp.float32)
    yt_ref[1] = jnp.dot(xs, w_ev[:, H:], preferred_element_type=jnp.float32)


@functools.partial(
    pl.kernel,
    out_type=jax.ShapeDtypeStruct((NC, NP, H), jnp.float32),
    mesh=_mesh,
    scratch_types=[
        pltpu.VMEM((CPS, CH), jnp.int32),
        pltpu.VMEM((CPS, CH), jnp.int32),
        pltpu.VMEM((CPS * CH,), jnp.float32),
        pltpu.VMEM((CH, H), jnp.float32),
        pltpu.VMEM((CH, H), jnp.float32),
        pltpu.VMEM_SHARED((NP, H), jnp.float32),
        pltpu.SemaphoreType.DMA,
        pltpu.SemaphoreType.DMA,
    ],
    compiler_params=pltpu.CompilerParams(needs_layout_passes=False),
)
def _spmm_kernel(y0, y1, srcs, dsts, ws, out_hbm,
                 src_v, dst_v, w_v, buf_a, buf_b, agg_sh, sem_a, sem_b):
    cid = lax.axis_index("c")
    sid = lax.axis_index("s")
    zero = jnp.zeros((L,), jnp.float32)

    def zrow(r, carry):
        for j in range(H // L):
            buf_a[r, pl.ds(j * L, L)] = zero
        return carry

    lax.fori_loop(0, CH, zrow, None)
    off = 0
    while off < ROWS_PT:
        rows = min(CH, ROWS_PT - off)
        pltpu.sync_copy(buf_a.at[pl.ds(0, rows)],
                        agg_sh.at[pl.ds(sid * ROWS_PT + off, rows)])
        off += rows
    plsc.subcore_barrier()

    def gather(c, buf, sem):
        @pl.when(cid == 0)
        def _g0():
            pltpu.async_copy(y0.at[src_v.at[c]], buf, sem)

        @pl.when(cid == 1)
        def _g1():
            pltpu.async_copy(y1.at[src_v.at[c]], buf, sem)

    def sblock(sb, carry):
        pltpu.sync_copy(srcs.at[sid, sb], src_v)
        pltpu.sync_copy(dsts.at[sid, sb], dst_v)
        pltpu.sync_copy(ws.at[sid, sb], w_v)
        gather(0, buf_a, sem_a)

        def grp(g, carry2):
            for b in range(2):
                c = g * 2 + b
                buf, sem = (buf_a, sem_a) if b == 0 else (buf_b, sem_b)
                nbuf, nsem = (buf_b, sem_b) if b == 0 else (buf_a, sem_a)

                @pl.when(c + 1 < CPS)
                def _pref():
                    gather(c + 1, nbuf, nsem)

                pltpu.make_async_copy(y0.at[src_v.at[c]], buf, sem).wait()

                def scale(e, inner):
                    wv = plsc.load_gather(
                        w_v, [jnp.full((L,), c * CH + e, jnp.int32)])
                    for j in range(H // L):
                        buf[e, pl.ds(j * L, L)] = buf[e, pl.ds(j * L, L)] * wv
                    return inner

                lax.fori_loop(0, CH, scale, None, unroll=5)
                pltpu.sync_copy(buf, agg_sh.at[dst_v.at[c]], add=True)
            return carry2

        lax.fori_loop(0, CPS // 2, grp, None)
        return carry

    lax.fori_loop(0, NSB, sblock, None)
    plsc.subcore_barrier()
    pltpu.sync_copy(agg_sh.at[pl.ds(sid * ROWS_PT, ROWS_PT)],
                    out_hbm.at[cid, pl.ds(sid * ROWS_PT, ROWS_PT)])


def _epilogue_body(agg_ref, yt_ref, dis_ref, bias_ref, lw_ref, lb_ref, out_ref):
    disf = dis_ref[...]
    h0 = jnp.maximum(
        disf * agg_ref[0] + (2.0 * disf) * yt_ref[0] + bias_ref[0:1, :], 0.0)
    h1 = jnp.maximum(
        disf * agg_ref[1] + (2.0 * disf) * yt_ref[1] + bias_ref[1:2, :], 0.0)
    h = jnp.concatenate([h0, h1], axis=1)
    out = jnp.dot(h, lw_ref[...], preferred_element_type=jnp.float32)
    out_ref[...] = out[:, 0:1] + lb_ref[0, 0]


def kernel(x, edge_index, edge_weight, W, conv_bias,
           lstm_w_ih, lstm_w_hh, lstm_b_ih, lstm_b_hh, lin_w, lin_b):
    src = edge_index[0].astype(jnp.int32)
    dst = edge_index[1].astype(jnp.int32)
    ew = edge_weight.astype(jnp.float32)
    x_p = jnp.pad(x, ((0, NP - N), (0, 0)))

    dst_a = jnp.full((EA,), N, jnp.int32).at[:E].set(dst)
    ew_a = jnp.zeros((EA,), jnp.float32).at[:E].set(ew)
    deg_parts = _deg_kernel(dst_a.reshape(NW, NCHA, CHA),
                            ew_a.reshape(NW, NCHA * CHA))

    b = (lstm_b_ih + lstm_b_hh).reshape(1, 4 * D)
    yt, dis = pl.pallas_call(
        _prologue_body,
        out_shape=[
            jax.ShapeDtypeStruct((2, NP, H), jnp.float32),
            jax.ShapeDtypeStruct((NP, 1), jnp.float32),
        ],
    )(W, lstm_w_ih, b, deg_parts, x_p)

    agg = _spmm_kernel(yt[0], yt[1],
                       src.reshape(NS, NSB, CPS, CH),
                       dst.reshape(NS, NSB, CPS, CH),
                       ew.reshape(NS, NSB, CPS * CH))

    out = pl.pallas_call(
        _epilogue_body,
        out_shape=jax.ShapeDtypeStruct((NP, 1), jnp.float32),
    )(agg, yt, dis, conv_bias.reshape(2, H),
      jnp.zeros((D, H), jnp.float32).at[:, 0].set(lin_w[0]),
      lin_b.reshape(1, 1))
    return out[:N]


# speedup vs baseline: 13.7710x; 1.0035x over previous
import functools

import jax
import jax.numpy as jnp
from jax import lax
from jax.experimental import pallas as pl
from jax.experimental.pallas import tpu as pltpu
from jax.experimental.pallas import tpu_sc as plsc

N = 10000
NP = 10240
D = 256
H = 128
E = 160000
NC, NS, L = 2, 16, 16
NW = NC * NS
CHA = 112
NCHA = 45
EPT_A = CHA * NCHA
EA = NW * EPT_A
EPT_B = E // NS
CH = 100
NCHUNK = EPT_B // CH
CPS = 20
NSB = NCHUNK // CPS
ROWS_PT = NP // NS

_mesh = plsc.VectorSubcoreMesh(
    core_axis_name="c", subcore_axis_name="s", num_cores=NC, num_subcores=NS)


@functools.partial(
    pl.kernel,
    out_type=jax.ShapeDtypeStruct((NC, NP, L), jnp.float32),
    mesh=_mesh,
    scratch_types=[
        pltpu.VMEM((NCHA, CHA), jnp.int32),
        pltpu.VMEM((NCHA * CHA,), jnp.float32),
        pltpu.VMEM((CHA, L), jnp.float32),
        pltpu.VMEM_SHARED((NP, L), jnp.float32),
    ],
    compiler_params=pltpu.CompilerParams(needs_layout_passes=False),
)
def _deg_kernel(dst_hbm, w_hbm, out_hbm, dst_v, w_v, wide, deg_sh):
    cid = lax.axis_index("c")
    sid = lax.axis_index("s")
    wid = sid * NC + cid
    pltpu.sync_copy(dst_hbm.at[wid], dst_v)
    pltpu.sync_copy(w_hbm.at[wid], w_v)
    zero = jnp.zeros((L,), jnp.float32)

    def zrow(r, carry):
        wide[r] = zero
        return carry

    lax.fori_loop(0, CHA, zrow, None)
    for k in range(ROWS_PT // CHA + 1):
        rows = min(CHA, ROWS_PT - k * CHA)
        if rows > 0:
            pltpu.sync_copy(wide.at[pl.ds(0, rows)],
                            deg_sh.at[pl.ds(sid * ROWS_PT + k * CHA, rows)])
    plsc.subcore_barrier()

    def chunk(c, carry):
        def fill(e, inner):
            wv = plsc.load_gather(w_v, [jnp.full((L,), c * CHA + e, jnp.int32)])
            wide[e] = wv
            return inner

        lax.fori_loop(0, CHA, fill, None)
        pltpu.sync_copy(wide, deg_sh.at[dst_v.at[c]], add=True)
        return carry

    lax.fori_loop(0, NCHA, chunk, None)
    plsc.subcore_barrier()
    pltpu.sync_copy(deg_sh.at[pl.ds(sid * ROWS_PT, ROWS_PT)],
                    out_hbm.at[cid, pl.ds(sid * ROWS_PT, ROWS_PT)])


def _prologue_body(w_ref, wih_ref, b_ref, degp_ref, x_ref, yt_ref, dis_ref):
    gates = lax.dot_general(
        w_ref[...], wih_ref[...], (((1,), (1,)), ((), ())),
        preferred_element_type=jnp.float32) + b_ref[...]
    i_ = jax.nn.sigmoid(gates[:, 0 * D:1 * D])
    g_ = jnp.tanh(gates[:, 2 * D:3 * D])
    o_ = jax.nn.sigmoid(gates[:, 3 * D:4 * D])
    w_ev = o_ * jnp.tanh(i_ * g_)
    deg = degp_ref[0, :, 0:1] + degp_ref[1, :, 0:1] + 2.0
    dis = lax.rsqrt(deg)
    dis_ref[...] = dis
    xs = x_ref[...] * dis
    yt_ref[0] = jnp.dot(xs, w_ev[:, :H], preferred_element_type=jnp.float32)
    yt_ref[1] = jnp.dot(xs, w_ev[:, H:], preferred_element_type=jnp.float32)


@functools.partial(
    pl.kernel,
    out_type=jax.ShapeDtypeStruct((NC, NP, H), jnp.float32),
    mesh=_mesh,
    scratch_types=[
        pltpu.VMEM((CPS, CH), jnp.int32),
        pltpu.VMEM((CPS, CH), jnp.int32),
        pltpu.VMEM((CPS * CH,), jnp.float32),
        pltpu.VMEM((CH, H), jnp.float32),
        pltpu.VMEM((CH, H), jnp.float32),
        pltpu.VMEM_SHARED((NP, H), jnp.float32),
        pltpu.SemaphoreType.DMA,
        pltpu.SemaphoreType.DMA,
        pltpu.SemaphoreType.DMA,
        pltpu.SemaphoreType.DMA,
    ],
    compiler_params=pltpu.CompilerParams(needs_layout_passes=False),
)
def _spmm_kernel(y0, y1, srcs, dsts, ws, out_hbm,
                 src_v, dst_v, w_v, buf_a, buf_b, agg_sh,
                 sem_a, sem_b, ssem_a, ssem_b):
    cid = lax.axis_index("c")
    sid = lax.axis_index("s")
    zero = jnp.zeros((L,), jnp.float32)

    def zrow(r, carry):
        for j in range(H // L):
            buf_a[r, pl.ds(j * L, L)] = zero
        return carry

    lax.fori_loop(0, CH, zrow, None)
    off = 0
    while off < ROWS_PT:
        rows = min(CH, ROWS_PT - off)
        pltpu.sync_copy(buf_a.at[pl.ds(0, rows)],
                        agg_sh.at[pl.ds(sid * ROWS_PT + off, rows)])
        off += rows
    plsc.subcore_barrier()

    def gather(c, buf, sem):
        @pl.when(cid == 0)
        def _g0():
            pltpu.async_copy(y0.at[src_v.at[c]], buf, sem)

        @pl.when(cid == 1)
        def _g1():
            pltpu.async_copy(y1.at[src_v.at[c]], buf, sem)

    def sblock(sb, carry):
        pltpu.sync_copy(srcs.at[sid, sb], src_v)
        pltpu.sync_copy(dsts.at[sid, sb], dst_v)
        pltpu.sync_copy(ws.at[sid, sb], w_v)
        gather(0, buf_a, sem_a)

        def grp(g, carry2):
            for b in range(2):
                c = g * 2 + b
                buf, sem, ssem = (buf_a, sem_a, ssem_a) if b == 0 else (
                    buf_b, sem_b, ssem_b)
                nbuf, nsem, nssem = (buf_b, sem_b, ssem_b) if b == 0 else (
                    buf_a, sem_a, ssem_a)

                @pl.when(c + 1 < CPS)
                def _pref():
                    @pl.when(c >= 1)
                    def _dr():
                        pltpu.make_async_copy(
                            nbuf, agg_sh.at[dst_v.at[c - 1]], nssem).wait()

                    gather(c + 1, nbuf, nsem)

                pltpu.make_async_copy(y0.at[src_v.at[c]], buf, sem).wait()

                def scale(e, inner):
                    wv = plsc.load_gather(
                        w_v, [jnp.full((L,), c * CH + e, jnp.int32)])
                    for j in range(H // L):
                        buf[e, pl.ds(j * L, L)] = buf[e, pl.ds(j * L, L)] * wv
                    return inner

                lax.fori_loop(0, CH, scale, None, unroll=5)
                pltpu.async_copy(buf, agg_sh.at[dst_v.at[c]], ssem, add=True)
            return carry2

        lax.fori_loop(0, CPS // 2, grp, None)
        pltpu.make_async_copy(
            buf_a, agg_sh.at[dst_v.at[CPS - 2]], ssem_a).wait()
        pltpu.make_async_copy(
            buf_b, agg_sh.at[dst_v.at[CPS - 1]], ssem_b).wait()
        return carry

    lax.fori_loop(0, NSB, sblock, None)
    plsc.subcore_barrier()
    pltpu.sync_copy(agg_sh.at[pl.ds(sid * ROWS_PT, ROWS_PT)],
                    out_hbm.at[cid, pl.ds(sid * ROWS_PT, ROWS_PT)])


def _epilogue_body(agg_ref, yt_ref, dis_ref, bias_ref, lw_ref, lb_ref, out_ref):
    disf = dis_ref[...]
    h0 = jnp.maximum(
        disf * agg_ref[0] + (2.0 * disf) * yt_ref[0] + bias_ref[0:1, :], 0.0)
    h1 = jnp.maximum(
        disf * agg_ref[1] + (2.0 * disf) * yt_ref[1] + bias_ref[1:2, :], 0.0)
    h = jnp.concatenate([h0, h1], axis=1)
    out = jnp.dot(h, lw_ref[...], preferred_element_type=jnp.float32)
    out_ref[...] = out[:, 0:1] + lb_ref[0, 0]


def kernel(x, edge_index, edge_weight, W, conv_bias,
           lstm_w_ih, lstm_w_hh, lstm_b_ih, lstm_b_hh, lin_w, lin_b):
    src = edge_index[0].astype(jnp.int32)
    dst = edge_index[1].astype(jnp.int32)
    ew = edge_weight.astype(jnp.float32)
    x_p = jnp.pad(x, ((0, NP - N), (0, 0)))

    dst_a = jnp.full((EA,), N, jnp.int32).at[:E].set(dst)
    ew_a = jnp.zeros((EA,), jnp.float32).at[:E].set(ew)
    deg_parts = _deg_kernel(dst_a.reshape(NW, NCHA, CHA),
                            ew_a.reshape(NW, NCHA * CHA))

    b = (lstm_b_ih + lstm_b_hh).reshape(1, 4 * D)
    yt, dis = pl.pallas_call(
        _prologue_body,
        out_shape=[
            jax.ShapeDtypeStruct((2, NP, H), jnp.float32),
            jax.ShapeDtypeStruct((NP, 1), jnp.float32),
        ],
    )(W, lstm_w_ih, b, deg_parts, x_p)

    agg = _spmm_kernel(yt[0], yt[1],
                       src.reshape(NS, NSB, CPS, CH),
                       dst.reshape(NS, NSB, CPS, CH),
                       ew.reshape(NS, NSB, CPS * CH))

    out = pl.pallas_call(
        _epilogue_body,
        out_shape=jax.ShapeDtypeStruct((NP, 1), jnp.float32),
    )(agg, yt, dis, conv_bias.reshape(2, H),
      jnp.zeros((D, H), jnp.float32).at[:, 0].set(lin_w[0]),
      lin_b.reshape(1, 1))
    return out[:N]
